# baseline (speedup 1.0000x reference)
import functools

import jax
import jax.numpy as jnp
from jax import lax
from jax.experimental import pallas as pl
from jax.experimental.pallas import tpu as pltpu
from jax.experimental.pallas import tpu_sc as plsc

D = 32
T = 200
NC = 2
NS = 16
NW = NC * NS


V = 1000000
DP = D
BLK = 128
N_FULL = V // BLK
TAIL = V - N_FULL * BLK
PER_W_BLOCKS = N_FULL // NW
N_EXTRA = N_FULL - PER_W_BLOCKS * NW


def _transpose_kernel():
  mesh = plsc.VectorSubcoreMesh(
      core_axis_name="c", subcore_axis_name="s", num_cores=NC,
      num_subcores=NS)

  @functools.partial(
      pl.kernel,
      out_type=jax.ShapeDtypeStruct((V * DP,), jnp.float32),
      mesh=mesh,
      scratch_types=[
          pltpu.VMEM((2, D, BLK), jnp.float32),
          pltpu.VMEM((2, BLK * DP), jnp.float32),
          pltpu.SemaphoreType.DMA((2,)),
          pltpu.SemaphoreType.DMA((2,)),
      ],
      compiler_params=pltpu.CompilerParams(needs_layout_passes=False),
  )
  def k(temb_hbm, tail_hbm, out_hbm, in_v, tr_v, lsem, ssem):
    wid = lax.axis_index("s") * NC + lax.axis_index("c")
    row_ids = [lax.iota(jnp.int32, 16), lax.iota(jnp.int32, 16) + 16]

    def load(blk, s):
      pltpu.async_copy(
          temb_hbm.at[:, pl.ds(blk * BLK, BLK)], in_v.at[s], lsem.at[s])

    def drain_load(s):
      pltpu.make_async_copy(
          temb_hbm.at[:, pl.ds(0, BLK)], in_v.at[s], lsem.at[s]).wait()

    def drain_store(s):
      pltpu.make_async_copy(
          tr_v.at[s], out_hbm.at[pl.ds(0, BLK * DP)], ssem.at[s]).wait()

    def transpose_rows(s, n_rows):
      def tr_body(r4, carry):
        rl = r4 * 4
        cols = [jnp.full((16,), rl + j, jnp.int32) for j in range(4)]
        gs = [plsc.load_gather(in_v.at[s], [row_ids[h], cols[j]])
              for j in range(4) for h in range(2)]
        for j in range(4):
          for h in range(2):
            tr_v[s, pl.ds((rl + j) * DP + h * 16, 16)] = gs[j * 2 + h]
        return carry

      lax.fori_loop(0, n_rows // 4, tr_body, 0, unroll=2)

    def finish(blk, s, store_pending):
      drain_load(s)
      @pl.when(store_pending)
      def _():
        drain_store(s)
      transpose_rows(s, BLK)
      pltpu.async_copy(
          tr_v.at[s], out_hbm.at[pl.ds(blk * (BLK * DP), BLK * DP)],
          ssem.at[s])

    first = wid * PER_W_BLOCKS
    load(first, 0)

    def pair_body(p, carry):
      b0 = first + 2 * p
      load(b0 + 1, 1)
      finish(b0, 0, p > 0)
      @pl.when(p < PER_W_BLOCKS // 2 - 1)
      def _():
        load(b0 + 2, 0)
      finish(b0 + 1, 1, p > 0)
      return carry

    lax.fori_loop(0, PER_W_BLOCKS // 2, pair_body, 0)
    drain_store(0)
    drain_store(1)

    @pl.when(wid >= NW - N_EXTRA)
    def _():
      blk = NW * PER_W_BLOCKS + (wid - (NW - N_EXTRA))
      load(blk, 0)
      drain_load(0)
      transpose_rows(0, BLK)
      pltpu.sync_copy(
          tr_v.at[0], out_hbm.at[pl.ds(blk * (BLK * DP), BLK * DP)])

    @pl.when(wid == NW - 1)
    def _():
      pltpu.sync_copy(tail_hbm, tr_v.at[0, pl.ds(0, TAIL * DP)])
      pltpu.sync_copy(
          tr_v.at[0, pl.ds(0, TAIL * DP)],
          out_hbm.at[pl.ds(N_FULL * (BLK * DP), TAIL * DP)])

  return k


NSLOT = 8
AHEAD = 4
BB = 128
XBLK = BB * T
TILE = 8 * BB


def _gather_kernel(batch):
  n_total = batch * T
  n_tb = batch // BB
  tbs_per_w = n_tb // NW
  mesh = plsc.VectorSubcoreMesh(
      core_axis_name="c", subcore_axis_name="s", num_cores=NC,
      num_subcores=NS)

  @functools.partial(
      pl.kernel,
      out_type=jax.ShapeDtypeStruct((n_total * D,), jnp.float32),
      mesh=mesh,
      scratch_types=[
          pltpu.VMEM((T, D), jnp.float32),
          pltpu.VMEM((XBLK,), jnp.int32),
          pltpu.VMEM((NSLOT, BB), jnp.int32),
          pltpu.VMEM((NSLOT, BB, DP), jnp.float32),
          pltpu.VMEM((NSLOT, 4 * TILE), jnp.float32),
          pltpu.SemaphoreType.DMA((NSLOT,)),
          pltpu.SemaphoreType.DMA((NSLOT,)),
      ],
      compiler_params=pltpu.CompilerParams(
          use_tc_tiling_on_sc=False, needs_layout_passes=False),
  )
  def k(idx_hbm, emb_hbm, pos_hbm, out_hbm, pos_v, xblk_v, idx_v, rows_v,
        out_v, gsem, ssem):
    wid = lax.axis_index("s") * NC + lax.axis_index("c")
    iota_t = lax.iota(jnp.int32, 16) * T
    iota = lax.iota(jnp.int32, 16)
    pltpu.sync_copy(pos_hbm, pos_v)

    def build_and_fire(t, s):
      tvec = jnp.full((16,), t, jnp.int32)
      ivs = [plsc.load_gather(xblk_v, [iota_t + (tvec + h * 16 * T)])
             for h in range(8)]
      for h in range(8):
        idx_v[s, pl.ds(h * 16, 16)] = ivs[h]
      pltpu.async_copy(emb_hbm.at[idx_v.at[s]], rows_v.at[s], gsem.at[s])

    def drain_gather(s):
      pltpu.make_async_copy(
          emb_hbm.at[pl.ds(0, BB)], rows_v.at[s], gsem.at[s]).wait()

    def drain_store(s):
      for _ in range(4):
        pltpu.make_async_copy(
            out_v.at[s, pl.ds(0, TILE)], out_hbm.at[pl.ds(0, TILE)],
            ssem.at[s]).wait()

    def finish(t, tb, s, store_pending):
      drain_gather(s)
      @pl.when(store_pending)
      def _():
        drain_store(s)
      tvec = jnp.full((16,), t, jnp.int32)

      def cbody(c, carry2):
        cvec = jnp.full((16,), c, jnp.int32)
        p = plsc.load_gather(pos_v, [tvec, cvec])
        gs = [plsc.load_gather(rows_v.at[s], [iota + h * 16, cvec])
              for h in range(8)]
        for h in range(8):
          out_v[s, pl.ds(c * BB + h * 16, 16)] = gs[h] + p
        return carry2

      lax.fori_loop(0, D, cbody, 0, unroll=4)
      for tc in range(4):
        pltpu.async_copy(
            out_v.at[s, pl.ds(tc * TILE, TILE)],
            out_hbm.at[pl.ds(t * (batch * D) + tc * (batch * 8) + tb * TILE,
                             TILE)],
            ssem.at[s])

    def tb_body(j, carry):
      tb = wid * tbs_per_w + j
      pltpu.sync_copy(idx_hbm.at[pl.ds(tb * XBLK, XBLK)], xblk_v)
      for u in range(AHEAD):
        build_and_fire(u, u)

      def group_body(p, carry2):
        t0 = p * NSLOT
        for b in range(NSLOT):
          t = t0 + b
          nxt = t + AHEAD
          if b < NSLOT - AHEAD:
            build_and_fire(nxt, (nxt) % NSLOT)
          else:
            @pl.when(p < T // NSLOT - 1)
            def _():
              build_and_fire(nxt, (nxt) % NSLOT)
          finish(t, tb, b, p > 0)
        return carry2

      lax.fori_loop(0, T // NSLOT, group_body, 0)
      for s in range(NSLOT):
        drain_store(s)
      return carry

    lax.fori_loop(0, tbs_per_w, tb_body, 0)

  return k


def kernel(x, embedding, pos_encoding):
  b, t = x.shape
  n_total = b * t
  tail = jnp.pad(embedding[V - TAIL:, :], ((0, 0), (0, DP - D)))
  tail = tail.reshape(TAIL * DP)
  table = _transpose_kernel()(embedding.T, tail)
  out = _gather_kernel(b)(
      x.reshape(n_total), table.reshape(V, DP), pos_encoding)
  out5 = out.reshape(T, D // 8, b // 128, 8, 128)
  return out5.transpose(2, 4, 0, 1, 3).reshape(b, t, D)


# speedup vs baseline: 1.0057x; 1.0057x over previous
import functools

import jax
import jax.numpy as jnp
from jax import lax
from jax.experimental import pallas as pl
from jax.experimental.pallas import tpu as pltpu
from jax.experimental.pallas import tpu_sc as plsc

D = 32
T = 200
NC = 2
NS = 16
NW = NC * NS


V = 1000000
DP = D
BLK = 128
N_FULL = V // BLK
TAIL = V - N_FULL * BLK
PER_W_BLOCKS = N_FULL // NW
N_EXTRA = N_FULL - PER_W_BLOCKS * NW


def _transpose_kernel():
  mesh = plsc.VectorSubcoreMesh(
      core_axis_name="c", subcore_axis_name="s", num_cores=NC,
      num_subcores=NS)

  @functools.partial(
      pl.kernel,
      out_type=jax.ShapeDtypeStruct((V * DP,), jnp.float32),
      mesh=mesh,
      scratch_types=[
          pltpu.VMEM((2, D, BLK), jnp.float32),
          pltpu.VMEM((2, BLK * DP), jnp.float32),
          pltpu.SemaphoreType.DMA((2,)),
          pltpu.SemaphoreType.DMA((2,)),
      ],
      compiler_params=pltpu.CompilerParams(needs_layout_passes=False),
  )
  def k(temb_hbm, tail_hbm, out_hbm, in_v, tr_v, lsem, ssem):
    wid = lax.axis_index("s") * NC + lax.axis_index("c")
    row_ids = [lax.iota(jnp.int32, 16), lax.iota(jnp.int32, 16) + 16]

    def load(blk, s):
      pltpu.async_copy(
          temb_hbm.at[:, pl.ds(blk * BLK, BLK)], in_v.at[s], lsem.at[s])

    def drain_load(s):
      pltpu.make_async_copy(
          temb_hbm.at[:, pl.ds(0, BLK)], in_v.at[s], lsem.at[s]).wait()

    def drain_store(s):
      pltpu.make_async_copy(
          tr_v.at[s], out_hbm.at[pl.ds(0, BLK * DP)], ssem.at[s]).wait()

    def transpose_rows(s, n_rows):
      def tr_body(r4, carry):
        rl = r4 * 4
        cols = [jnp.full((16,), rl + j, jnp.int32) for j in range(4)]
        gs = [plsc.load_gather(in_v.at[s], [row_ids[h], cols[j]])
              for j in range(4) for h in range(2)]
        for j in range(4):
          for h in range(2):
            tr_v[s, pl.ds((rl + j) * DP + h * 16, 16)] = gs[j * 2 + h]
        return carry

      lax.fori_loop(0, n_rows // 4, tr_body, 0, unroll=2)

    def finish(blk, s, store_pending):
      drain_load(s)
      @pl.when(store_pending)
      def _():
        drain_store(s)
      transpose_rows(s, BLK)
      pltpu.async_copy(
          tr_v.at[s], out_hbm.at[pl.ds(blk * (BLK * DP), BLK * DP)],
          ssem.at[s])

    first = wid * PER_W_BLOCKS
    load(first, 0)

    def pair_body(p, carry):
      b0 = first + 2 * p
      load(b0 + 1, 1)
      finish(b0, 0, p > 0)
      @pl.when(p < PER_W_BLOCKS // 2 - 1)
      def _():
        load(b0 + 2, 0)
      finish(b0 + 1, 1, p > 0)
      return carry

    lax.fori_loop(0, PER_W_BLOCKS // 2, pair_body, 0)
    drain_store(0)
    drain_store(1)

    @pl.when(wid >= NW - N_EXTRA)
    def _():
      blk = NW * PER_W_BLOCKS + (wid - (NW - N_EXTRA))
      load(blk, 0)
      drain_load(0)
      transpose_rows(0, BLK)
      pltpu.sync_copy(
          tr_v.at[0], out_hbm.at[pl.ds(blk * (BLK * DP), BLK * DP)])

    @pl.when(wid == NW - 1)
    def _():
      pltpu.sync_copy(tail_hbm, tr_v.at[0, pl.ds(0, TAIL * DP)])
      pltpu.sync_copy(
          tr_v.at[0, pl.ds(0, TAIL * DP)],
          out_hbm.at[pl.ds(N_FULL * (BLK * DP), TAIL * DP)])

  return k


NSLOT = 8
AHEAD = 4
BB = 128
XBLK = BB * T
TILE = 8 * BB


def _gather_kernel(batch):
  n_total = batch * T
  n_tb = batch // BB
  tbs_per_w = n_tb // NW
  mesh = plsc.VectorSubcoreMesh(
      core_axis_name="c", subcore_axis_name="s", num_cores=NC,
      num_subcores=NS)

  @functools.partial(
      pl.kernel,
      out_type=jax.ShapeDtypeStruct((n_total * D,), jnp.float32),
      mesh=mesh,
      scratch_types=[
          pltpu.VMEM((T, D), jnp.float32),
          pltpu.VMEM((XBLK,), jnp.int32),
          pltpu.VMEM((NSLOT, BB), jnp.int32),
          pltpu.VMEM((NSLOT, BB, DP), jnp.float32),
          pltpu.VMEM((NSLOT, 4 * TILE), jnp.float32),
          pltpu.SemaphoreType.DMA((NSLOT,)),
          pltpu.SemaphoreType.DMA((NSLOT,)),
      ],
      compiler_params=pltpu.CompilerParams(
          use_tc_tiling_on_sc=False, needs_layout_passes=False),
  )
  def k(idx_hbm, emb_hbm, pos_hbm, out_hbm, pos_v, xblk_v, idx_v, rows_v,
        out_v, gsem, ssem):
    wid = lax.axis_index("s") * NC + lax.axis_index("c")
    iota_t = lax.iota(jnp.int32, 16) * T
    iota = lax.iota(jnp.int32, 16)
    pltpu.sync_copy(pos_hbm, pos_v)

    def build_and_fire(t, s):
      tvec = jnp.full((16,), t, jnp.int32)
      ivs = [plsc.load_gather(xblk_v, [iota_t + (tvec + h * 16 * T)])
             for h in range(8)]
      for h in range(8):
        idx_v[s, pl.ds(h * 16, 16)] = ivs[h]
      pltpu.async_copy(emb_hbm.at[idx_v.at[s]], rows_v.at[s], gsem.at[s])

    def drain_gather(s):
      pltpu.make_async_copy(
          emb_hbm.at[pl.ds(0, BB)], rows_v.at[s], gsem.at[s]).wait()

    def drain_store(s):
      for _ in range(4):
        pltpu.make_async_copy(
            out_v.at[s, pl.ds(0, TILE)], out_hbm.at[pl.ds(0, TILE)],
            ssem.at[s]).wait()

    def finish(t, tb, s, store_pending):
      drain_gather(s)
      @pl.when(store_pending)
      def _():
        drain_store(s)
      tvec = jnp.full((16,), t, jnp.int32)

      def cbody(c, carry2):
        cvec = jnp.full((16,), c, jnp.int32)
        p = plsc.load_gather(pos_v, [tvec, cvec])
        gs = [plsc.load_gather(rows_v.at[s], [iota + h * 16, cvec])
              for h in range(8)]
        for h in range(8):
          out_v[s, pl.ds(c * BB + h * 16, 16)] = gs[h] + p
        return carry2

      lax.fori_loop(0, D, cbody, 0, unroll=2)
      for tc in range(4):
        pltpu.async_copy(
            out_v.at[s, pl.ds(tc * TILE, TILE)],
            out_hbm.at[pl.ds(t * (batch * D) + tc * (batch * 8) + tb * TILE,
                             TILE)],
            ssem.at[s])

    def tb_body(j, carry):
      tb = wid * tbs_per_w + j
      pltpu.sync_copy(idx_hbm.at[pl.ds(tb * XBLK, XBLK)], xblk_v)
      for u in range(AHEAD):
        build_and_fire(u, u)

      def group_body(p, carry2):
        t0 = p * NSLOT
        for b in range(NSLOT):
          t = t0 + b
          nxt = t + AHEAD
          if b < NSLOT - AHEAD:
            build_and_fire(nxt, (nxt) % NSLOT)
          else:
            @pl.when(p < T // NSLOT - 1)
            def _():
              build_and_fire(nxt, (nxt) % NSLOT)
          finish(t, tb, b, p > 0)
        return carry2

      lax.fori_loop(0, T // NSLOT, group_body, 0)
      for s in range(NSLOT):
        drain_store(s)
      return carry

    lax.fori_loop(0, tbs_per_w, tb_body, 0)

  return k


def kernel(x, embedding, pos_encoding):
  b, t = x.shape
  n_total = b * t
  tail = jnp.pad(embedding[V - TAIL:, :], ((0, 0), (0, DP - D)))
  tail = tail.reshape(TAIL * DP)
  table = _transpose_kernel()(embedding.T, tail)
  out = _gather_kernel(b)(
      x.reshape(n_total), table.reshape(V, DP), pos_encoding)
  out5 = out.reshape(T, D // 8, b // 128, 8, 128)
  return out5.transpose(2, 4, 0, 1, 3).reshape(b, t, D)
